# Initial kernel scaffold; baseline (speedup 1.0000x reference)
import jax
import jax.numpy as jnp
from jax.experimental import pallas as pl


def kernel(hidden_states, positions, input_ln_w, post_ln_w, W_qkv_a, q_a_ln_w, W_q_b, kv_a_ln_w, W_kv_b, W_o, W_router, e_bias, W_g, W_u, W_d, Ws_g, Ws_u, Ws_d):
    raise NotImplementedError("write your pallas kernel here")


# speedup vs baseline: 1.4563x; 1.4563x over previous
import functools

import jax
import jax.numpy as jnp
from jax.experimental import pallas as pl
from jax.experimental.pallas import tpu as pltpu

T = 2048
D = 2048
H = 16
DQN = 128
DR = 64
DV = 128
QLR = 1536
KVLR = 512
E = 64
K = 8
F = 128
EPS = 1e-06
SCALE = (DQN + DR) ** -0.5
RSF = 2.5
CAP = int(T * K / E * 2)

BM = 256


def _rms_bf16(x, w):
    v = jnp.mean(x * x, axis=-1, keepdims=True)
    return (x * jax.lax.rsqrt(v + EPS) * w).astype(jnp.bfloat16)


def _mm_rms_body(x_ref, wln_ref, w_ref, o_ref):
    xn = _rms_bf16(x_ref[...], wln_ref[...])
    o_ref[...] = jnp.dot(xn, w_ref[...].astype(jnp.bfloat16),
                         preferred_element_type=jnp.float32)


def _mm_rms(x, wln, w):
    t, k = x.shape
    n = w.shape[1]
    return pl.pallas_call(
        _mm_rms_body,
        grid=(t // BM,),
        in_specs=[
            pl.BlockSpec((BM, k), lambda i: (i, 0)),
            pl.BlockSpec((1, k), lambda i: (0, 0)),
            pl.BlockSpec((k, n), lambda i: (0, 0)),
        ],
        out_specs=pl.BlockSpec((BM, n), lambda i: (i, 0)),
        out_shape=jax.ShapeDtypeStruct((t, n), jnp.float32),
    )(x, wln.reshape(1, k), w)


def _attn_body(qn_ref, qe_ref, qo_ref, kn_ref, ke_ref, ko_ref, v_ref,
               cos_ref, sin_ref, o_ref):
    qb = pl.program_id(1)
    q0 = qb * BM
    cq = cos_ref[pl.ds(q0, BM), :]
    sq = sin_ref[pl.ds(q0, BM), :]
    qe = qe_ref[0]
    qo = qo_ref[0]
    qf = jnp.concatenate(
        [qn_ref[0], qe * cq - qo * sq, qo * cq + qe * sq], axis=1
    ).astype(jnp.bfloat16)
    ke = ke_ref[...]
    ko = ko_ref[...]
    ck = cos_ref[...]
    sk = sin_ref[...]
    kf = jnp.concatenate(
        [kn_ref[0], ke * ck - ko * sk, ko * ck + ke * sk], axis=1
    ).astype(jnp.bfloat16)
    s = jax.lax.dot_general(qf, kf, (((1,), (1,)), ((), ())),
                            preferred_element_type=jnp.float32) * SCALE
    row = q0 + jax.lax.broadcasted_iota(jnp.int32, (BM, T), 0)
    col = jax.lax.broadcasted_iota(jnp.int32, (BM, T), 1)
    s = jnp.where(col <= row, s, -1e30)
    m = jnp.max(s, axis=1, keepdims=True)
    p = jnp.exp(s - m)
    l = jnp.sum(p, axis=1, keepdims=True)
    o = jnp.dot(p.astype(jnp.bfloat16), v_ref[0].astype(jnp.bfloat16),
                preferred_element_type=jnp.float32) / l
    o_ref[0] = o


def _attention(qn, qe, qo, kn, ke, ko, v, cos, sin):
    return pl.pallas_call(
        _attn_body,
        grid=(H, T // BM),
        in_specs=[
            pl.BlockSpec((1, BM, DQN), lambda h, q: (h, q, 0)),
            pl.BlockSpec((1, BM, DR // 2), lambda h, q: (h, q, 0)),
            pl.BlockSpec((1, BM, DR // 2), lambda h, q: (h, q, 0)),
            pl.BlockSpec((1, T, DQN), lambda h, q: (h, 0, 0)),
            pl.BlockSpec((T, DR // 2), lambda h, q: (0, 0)),
            pl.BlockSpec((T, DR // 2), lambda h, q: (0, 0)),
            pl.BlockSpec((1, T, DV), lambda h, q: (h, 0, 0)),
            pl.BlockSpec((T, DR // 2), lambda h, q: (0, 0)),
            pl.BlockSpec((T, DR // 2), lambda h, q: (0, 0)),
        ],
        out_specs=pl.BlockSpec((1, BM, DV), lambda h, q: (h, q, 0)),
        out_shape=jax.ShapeDtypeStruct((H, T, DV), jnp.float32),
    )(qn, qe, qo, kn, ke, ko, v, cos, sin)


def _wo_body(o_ref, w_ref, hid_ref, h_ref):
    h_ref[...] = hid_ref[...] + jnp.dot(
        o_ref[...].astype(jnp.bfloat16), w_ref[...].astype(jnp.bfloat16),
        preferred_element_type=jnp.float32)


def _wo_res(o_cat, w_o, hidden):
    return pl.pallas_call(
        _wo_body,
        grid=(T // BM,),
        in_specs=[
            pl.BlockSpec((BM, H * DV), lambda i: (i, 0)),
            pl.BlockSpec((H * DV, D), lambda i: (0, 0)),
            pl.BlockSpec((BM, D), lambda i: (i, 0)),
        ],
        out_specs=pl.BlockSpec((BM, D), lambda i: (i, 0)),
        out_shape=jax.ShapeDtypeStruct((T, D), jnp.float32),
    )(o_cat, w_o, hidden)


def _router_body(h_ref, wln_ref, wr_ref, eb_ref, x2_ref, sel_ref, w_ref):
    h = h_ref[...]
    v = jnp.mean(h * h, axis=-1, keepdims=True)
    x2 = h * jax.lax.rsqrt(v + EPS) * wln_ref[...]
    x2_ref[...] = x2.astype(jnp.bfloat16)
    logits = jnp.dot(x2, wr_ref[...], preferred_element_type=jnp.float32)
    sc = jax.nn.sigmoid(logits)
    b = sc + eb_ref[...]
    idx = jax.lax.broadcasted_iota(jnp.int32, (BM, E), 1)
    sel = jnp.zeros((BM, E), jnp.float32)
    for _ in range(K):
        m = jnp.max(b, axis=1, keepdims=True)
        am = jnp.min(jnp.where(b == m, idx, E), axis=1, keepdims=True)
        pick = idx == am
        sel = jnp.where(pick, 1.0, sel)
        b = jnp.where(pick, -jnp.inf, b)
    sw = sel * sc
    denom = jnp.sum(sw, axis=1, keepdims=True) + 1e-20
    sel_ref[...] = sel
    w_ref[...] = sw / denom * RSF


def _router(h, post_ln_w, w_router, e_bias):
    return pl.pallas_call(
        _router_body,
        grid=(T // BM,),
        in_specs=[
            pl.BlockSpec((BM, D), lambda i: (i, 0)),
            pl.BlockSpec((1, D), lambda i: (0, 0)),
            pl.BlockSpec((D, E), lambda i: (0, 0)),
            pl.BlockSpec((1, E), lambda i: (0, 0)),
        ],
        out_specs=[
            pl.BlockSpec((BM, D), lambda i: (i, 0)),
            pl.BlockSpec((BM, E), lambda i: (i, 0)),
            pl.BlockSpec((BM, E), lambda i: (i, 0)),
        ],
        out_shape=[
            jax.ShapeDtypeStruct((T, D), jnp.bfloat16),
            jax.ShapeDtypeStruct((T, E), jnp.float32),
            jax.ShapeDtypeStruct((T, E), jnp.float32),
        ],
    )(h, post_ln_w.reshape(1, D), w_router, e_bias.reshape(1, E))


def _cap_body(sel_ref, w_ref, kw_ref, carry_ref):
    i = pl.program_id(0)

    @pl.when(i == 0)
    def _():
        carry_ref[...] = jnp.zeros_like(carry_ref)

    sel = sel_ref[...]
    r = jax.lax.broadcasted_iota(jnp.int32, (BM, BM), 0)
    c = jax.lax.broadcasted_iota(jnp.int32, (BM, BM), 1)
    tril = (r >= c).astype(jnp.bfloat16)
    cs = jnp.dot(tril, sel.astype(jnp.bfloat16),
                 preferred_element_type=jnp.float32)
    pie = cs - 1.0 + carry_ref[...]
    keep = jnp.where(pie < CAP, sel, 0.0)
    kw_ref[...] = w_ref[...] * keep
    carry_ref[...] += cs[BM - 1:BM, :]


def _capacity(sel, wfull):
    return pl.pallas_call(
        _cap_body,
        grid=(T // BM,),
        in_specs=[
            pl.BlockSpec((BM, E), lambda i: (i, 0)),
            pl.BlockSpec((BM, E), lambda i: (i, 0)),
        ],
        out_specs=pl.BlockSpec((BM, E), lambda i: (i, 0)),
        out_shape=jax.ShapeDtypeStruct((T, E), jnp.float32),
        scratch_shapes=[pltpu.VMEM((1, E), jnp.float32)],
    )(sel, wfull)


def _exp_body(x2_ref, wg_ref, wu_ref, kw_ref, hg_ref):
    e = pl.program_id(0)
    x2 = x2_ref[...]
    xg = jnp.dot(x2, wg_ref[0].astype(jnp.bfloat16),
                 preferred_element_type=jnp.float32)
    xu = jnp.dot(x2, wu_ref[0].astype(jnp.bfloat16),
                 preferred_element_type=jnp.float32)
    ohe = (jax.lax.broadcasted_iota(jnp.int32, (1, E), 1) == e
           ).astype(jnp.float32)
    kcol = jnp.sum(kw_ref[...] * ohe, axis=1, keepdims=True)
    hg = jax.nn.silu(xg) * xu * kcol
    hg_ref[...] = hg.astype(jnp.bfloat16)


def _experts_hg(x2b, w_g, w_u, keepw):
    return pl.pallas_call(
        _exp_body,
        grid=(E,),
        in_specs=[
            pl.BlockSpec((T, D), lambda e: (0, 0)),
            pl.BlockSpec((1, D, F), lambda e: (e, 0, 0)),
            pl.BlockSpec((1, D, F), lambda e: (e, 0, 0)),
            pl.BlockSpec((T, E), lambda e: (0, 0)),
        ],
        out_specs=pl.BlockSpec((T, F), lambda e: (0, e)),
        out_shape=jax.ShapeDtypeStruct((T, E * F), jnp.bfloat16),
    )(x2b, w_g, w_u, keepw)


def _final_body(hg_ref, wd_ref, h_ref, x2_ref, wsg_ref, wsu_ref, wsd_ref,
                o_ref):
    x2 = x2_ref[...]
    sg = jnp.dot(x2, wsg_ref[...].astype(jnp.bfloat16),
                 preferred_element_type=jnp.float32)
    su = jnp.dot(x2, wsu_ref[...].astype(jnp.bfloat16),
                 preferred_element_type=jnp.float32)
    hs = (jax.nn.silu(sg) * su).astype(jnp.bfloat16)
    shared = jnp.dot(hs, wsd_ref[...].astype(jnp.bfloat16),
                     preferred_element_type=jnp.float32)
    routed = jnp.dot(hg_ref[...], wd_ref[...],
                     preferred_element_type=jnp.float32)
    o_ref[...] = h_ref[...] + routed + shared


def _final(hg, wd_stack_bf16, h, x2b, ws_g, ws_u, ws_d):
    return pl.pallas_call(
        _final_body,
        grid=(T // BM,),
        in_specs=[
            pl.BlockSpec((BM, E * F), lambda i: (i, 0)),
            pl.BlockSpec((E * F, D), lambda i: (0, 0)),
            pl.BlockSpec((BM, D), lambda i: (i, 0)),
            pl.BlockSpec((BM, D), lambda i: (i, 0)),
            pl.BlockSpec((D, F), lambda i: (0, 0)),
            pl.BlockSpec((D, F), lambda i: (0, 0)),
            pl.BlockSpec((F, D), lambda i: (0, 0)),
        ],
        out_specs=pl.BlockSpec((BM, D), lambda i: (i, 0)),
        out_shape=jax.ShapeDtypeStruct((T, D), jnp.float32),
    )(hg, wd_stack_bf16, h, x2b, ws_g, ws_u, ws_d)


def kernel(hidden_states, positions, input_ln_w, post_ln_w, W_qkv_a,
           q_a_ln_w, W_q_b, kv_a_ln_w, W_kv_b, W_o, W_router, e_bias,
           W_g, W_u, W_d, Ws_g, Ws_u, Ws_d):
    inv = 1.0 / (10000.0 ** (jnp.arange(0, DR, 2, dtype=jnp.float32) / DR))
    ang = positions.astype(jnp.float32)[:, None] * inv[None, :]
    cos = jnp.cos(ang)
    sin = jnp.sin(ang)

    qkv = _mm_rms(hidden_states, input_ln_w, W_qkv_a)
    q_l = qkv[:, :QLR]
    kv_l = qkv[:, QLR:QLR + KVLR]
    k_pe = qkv[:, QLR + KVLR:]
    q = _mm_rms(q_l, q_a_ln_w, W_q_b)
    kv = _mm_rms(kv_l, kv_a_ln_w, W_kv_b)

    qh = q.reshape(T, H, DQN + DR).transpose(1, 0, 2)
    qn = qh[..., :DQN]
    qe = qh[..., DQN::2]
    qo = qh[..., DQN + 1::2]
    kvh = kv.reshape(T, H, DQN + DV).transpose(1, 0, 2)
    kn = kvh[..., :DQN]
    v = kvh[..., DQN:]
    ke = k_pe[:, 0::2]
    ko = k_pe[:, 1::2]

    o = _attention(qn, qe, qo, kn, ke, ko, v, cos, sin)
    o_cat = o.transpose(1, 0, 2).reshape(T, H * DV)
    h = _wo_res(o_cat, W_o, hidden_states)

    x2b, sel, wfull = _router(h, post_ln_w, W_router, e_bias)
    keepw = _capacity(sel, wfull)

    hg = _experts_hg(x2b, W_g, W_u, keepw)
    wd_stack = W_d.reshape(E * F, D).astype(jnp.bfloat16)
    out = _final(hg, wd_stack, h, x2b, Ws_g, Ws_u, Ws_d)
    return out


# speedup vs baseline: 2.2821x; 1.5671x over previous
import functools

import jax
import jax.numpy as jnp
from jax.experimental import pallas as pl
from jax.experimental.pallas import tpu as pltpu

T = 2048
D = 2048
H = 16
DQN = 128
DR = 64
DV = 128
QLR = 1536
KVLR = 512
E = 64
K = 8
F = 128
EPS = 1e-06
SCALE = (DQN + DR) ** -0.5
RSF = 2.5
CAP = int(T * K / E * 2)

BM = 256


def _rms_bf16(x, w):
    v = jnp.mean(x * x, axis=-1, keepdims=True)
    return (x * jax.lax.rsqrt(v + EPS) * w).astype(jnp.bfloat16)


def _mm_rms_body(x_ref, wln_ref, w_ref, o_ref):
    xn = _rms_bf16(x_ref[...], wln_ref[...])
    o_ref[...] = jnp.dot(xn, w_ref[...].astype(jnp.bfloat16),
                         preferred_element_type=jnp.float32)


def _mm_rms(x, wln, w, kwidth=None, colblk=0):
    t = x.shape[0]
    k = w.shape[0] if kwidth is None else kwidth
    n = w.shape[1]
    return pl.pallas_call(
        _mm_rms_body,
        grid=(t // BM,),
        in_specs=[
            pl.BlockSpec((BM, k), lambda i: (i, colblk)),
            pl.BlockSpec((1, k), lambda i: (0, 0)),
            pl.BlockSpec((k, n), lambda i: (0, 0)),
        ],
        out_specs=pl.BlockSpec((BM, n), lambda i: (i, 0)),
        out_shape=jax.ShapeDtypeStruct((t, n), jnp.float32),
    )(x, wln.reshape(1, k), w)


def _rope_i(x, c, s):
    rp = jnp.roll(x, 1, axis=1)
    rm = jnp.roll(x, -1, axis=1)
    lane = jax.lax.broadcasted_iota(jnp.int32, x.shape, 1)
    swap = jnp.where(lane % 2 == 0, -rm, rp)
    return x * c + swap * s


def _attn_body(q_ref, kpe_ref, kv_ref, cos_ref, sin_ref, o_ref):
    q0 = pl.program_id(1) * BM
    ck = cos_ref[...]
    sk = sin_ref[...]
    cq = cos_ref[pl.ds(q0, BM), :]
    sq = sin_ref[pl.ds(q0, BM), :]
    kpe = _rope_i(kpe_ref[...], ck, sk)
    qblk = q_ref[...]
    kvblk = kv_ref[...]
    row = q0 + jax.lax.broadcasted_iota(jnp.int32, (BM, T), 0)
    col = jax.lax.broadcasted_iota(jnp.int32, (BM, T), 1)
    for hh in range(2):
        qh = qblk[:, hh * 192:(hh + 1) * 192]
        qf = jnp.concatenate(
            [qh[:, :DQN], _rope_i(qh[:, DQN:], cq, sq)], axis=1
        ).astype(jnp.bfloat16)
        kn = kvblk[:, hh * 256:hh * 256 + DQN]
        v = kvblk[:, hh * 256 + DQN:(hh + 1) * 256]
        kf = jnp.concatenate([kn, kpe], axis=1).astype(jnp.bfloat16)
        s = jax.lax.dot_general(qf, kf, (((1,), (1,)), ((), ())),
                                preferred_element_type=jnp.float32) * SCALE
        s = jnp.where(col <= row, s, -1e30)
        m = jnp.max(s, axis=1, keepdims=True)
        p = jnp.exp(s - m)
        l = jnp.sum(p, axis=1, keepdims=True)
        o = jnp.dot(p.astype(jnp.bfloat16), v.astype(jnp.bfloat16),
                    preferred_element_type=jnp.float32) / l
        o_ref[:, hh * DV:(hh + 1) * DV] = o


def _attention(q, k_pe, kv, cos2, sin2):
    return pl.pallas_call(
        _attn_body,
        grid=(H // 2, T // BM),
        in_specs=[
            pl.BlockSpec((BM, 384), lambda h, q: (q, h)),
            pl.BlockSpec((T, DR), lambda h, q: (0, 0)),
            pl.BlockSpec((T, 512), lambda h, q: (0, h)),
            pl.BlockSpec((T, DR), lambda h, q: (0, 0)),
            pl.BlockSpec((T, DR), lambda h, q: (0, 0)),
        ],
        out_specs=pl.BlockSpec((BM, 2 * DV), lambda h, q: (q, h)),
        out_shape=jax.ShapeDtypeStruct((T, H * DV), jnp.float32),
    )(q, k_pe, kv, cos2, sin2)


def _wo_body(o_ref, w_ref, hid_ref, h_ref):
    h_ref[...] = hid_ref[...] + jnp.dot(
        o_ref[...].astype(jnp.bfloat16), w_ref[...].astype(jnp.bfloat16),
        preferred_element_type=jnp.float32)


def _wo_res(o_cat, w_o, hidden):
    return pl.pallas_call(
        _wo_body,
        grid=(T // BM,),
        in_specs=[
            pl.BlockSpec((BM, H * DV), lambda i: (i, 0)),
            pl.BlockSpec((H * DV, D), lambda i: (0, 0)),
            pl.BlockSpec((BM, D), lambda i: (i, 0)),
        ],
        out_specs=pl.BlockSpec((BM, D), lambda i: (i, 0)),
        out_shape=jax.ShapeDtypeStruct((T, D), jnp.float32),
    )(o_cat, w_o, hidden)


def _router_body(h_ref, wln_ref, wr_ref, eb_ref, x2_ref, sel_ref, w_ref):
    h = h_ref[...]
    v = jnp.mean(h * h, axis=-1, keepdims=True)
    x2 = h * jax.lax.rsqrt(v + EPS) * wln_ref[...]
    x2_ref[...] = x2.astype(jnp.bfloat16)
    logits = jnp.dot(x2, wr_ref[...], preferred_element_type=jnp.float32)
    sc = jax.nn.sigmoid(logits)
    b = sc + eb_ref[...]
    idx = jax.lax.broadcasted_iota(jnp.int32, (BM, E), 1)
    sel = jnp.zeros((BM, E), jnp.float32)
    for _ in range(K):
        m = jnp.max(b, axis=1, keepdims=True)
        am = jnp.min(jnp.where(b == m, idx, E), axis=1, keepdims=True)
        pick = idx == am
        sel = jnp.where(pick, 1.0, sel)
        b = jnp.where(pick, -jnp.inf, b)
    sw = sel * sc
    denom = jnp.sum(sw, axis=1, keepdims=True) + 1e-20
    sel_ref[...] = sel
    w_ref[...] = sw / denom * RSF


def _router(h, post_ln_w, w_router, e_bias):
    return pl.pallas_call(
        _router_body,
        grid=(T // BM,),
        in_specs=[
            pl.BlockSpec((BM, D), lambda i: (i, 0)),
            pl.BlockSpec((1, D), lambda i: (0, 0)),
            pl.BlockSpec((D, E), lambda i: (0, 0)),
            pl.BlockSpec((1, E), lambda i: (0, 0)),
        ],
        out_specs=[
            pl.BlockSpec((BM, D), lambda i: (i, 0)),
            pl.BlockSpec((BM, E), lambda i: (i, 0)),
            pl.BlockSpec((BM, E), lambda i: (i, 0)),
        ],
        out_shape=[
            jax.ShapeDtypeStruct((T, D), jnp.bfloat16),
            jax.ShapeDtypeStruct((T, E), jnp.float32),
            jax.ShapeDtypeStruct((T, E), jnp.float32),
        ],
    )(h, post_ln_w.reshape(1, D), w_router, e_bias.reshape(1, E))


def _cap_body(sel_ref, w_ref, kw_ref, carry_ref):
    i = pl.program_id(0)

    @pl.when(i == 0)
    def _():
        carry_ref[...] = jnp.zeros_like(carry_ref)

    sel = sel_ref[...]
    r = jax.lax.broadcasted_iota(jnp.int32, (BM, BM), 0)
    c = jax.lax.broadcasted_iota(jnp.int32, (BM, BM), 1)
    tril = (r >= c).astype(jnp.bfloat16)
    cs = jnp.dot(tril, sel.astype(jnp.bfloat16),
                 preferred_element_type=jnp.float32)
    pie = cs - 1.0 + carry_ref[...]
    keep = jnp.where(pie < CAP, sel, 0.0)
    kw_ref[...] = w_ref[...] * keep
    carry_ref[...] += cs[BM - 1:BM, :]


def _capacity(sel, wfull):
    return pl.pallas_call(
        _cap_body,
        grid=(T // BM,),
        in_specs=[
            pl.BlockSpec((BM, E), lambda i: (i, 0)),
            pl.BlockSpec((BM, E), lambda i: (i, 0)),
        ],
        out_specs=pl.BlockSpec((BM, E), lambda i: (i, 0)),
        out_shape=jax.ShapeDtypeStruct((T, E), jnp.float32),
        scratch_shapes=[pltpu.VMEM((1, E), jnp.float32)],
    )(sel, wfull)


def _exp_body(x2_ref, wg_ref, wu_ref, kw_ref, hg_ref):
    e = pl.program_id(0)
    x2 = x2_ref[...]
    xg = jnp.dot(x2, wg_ref[0].astype(jnp.bfloat16),
                 preferred_element_type=jnp.float32)
    xu = jnp.dot(x2, wu_ref[0].astype(jnp.bfloat16),
                 preferred_element_type=jnp.float32)
    ohe = (jax.lax.broadcasted_iota(jnp.int32, (1, E), 1) == e
           ).astype(jnp.float32)
    kcol = jnp.sum(kw_ref[...] * ohe, axis=1, keepdims=True)
    hg = jax.nn.silu(xg) * xu * kcol
    hg_ref[...] = hg.astype(jnp.bfloat16)


def _experts_hg(x2b, w_g, w_u, keepw):
    return pl.pallas_call(
        _exp_body,
        grid=(E,),
        in_specs=[
            pl.BlockSpec((T, D), lambda e: (0, 0)),
            pl.BlockSpec((1, D, F), lambda e: (e, 0, 0)),
            pl.BlockSpec((1, D, F), lambda e: (e, 0, 0)),
            pl.BlockSpec((T, E), lambda e: (0, 0)),
        ],
        out_specs=pl.BlockSpec((T, F), lambda e: (0, e)),
        out_shape=jax.ShapeDtypeStruct((T, E * F), jnp.bfloat16),
    )(x2b, w_g, w_u, keepw)


def _final_body(hg_ref, wd_ref, h_ref, x2_ref, wsg_ref, wsu_ref, wsd_ref,
                o_ref):
    x2 = x2_ref[...]
    sg = jnp.dot(x2, wsg_ref[...].astype(jnp.bfloat16),
                 preferred_element_type=jnp.float32)
    su = jnp.dot(x2, wsu_ref[...].astype(jnp.bfloat16),
                 preferred_element_type=jnp.float32)
    hs = (jax.nn.silu(sg) * su).astype(jnp.bfloat16)
    shared = jnp.dot(hs, wsd_ref[...].astype(jnp.bfloat16),
                     preferred_element_type=jnp.float32)
    routed = jnp.dot(hg_ref[...], wd_ref[...],
                     preferred_element_type=jnp.float32)
    o_ref[...] = h_ref[...] + routed + shared


def _final(hg, wd_stack_bf16, h, x2b, ws_g, ws_u, ws_d):
    return pl.pallas_call(
        _final_body,
        grid=(T // BM,),
        in_specs=[
            pl.BlockSpec((BM, E * F), lambda i: (i, 0)),
            pl.BlockSpec((E * F, D), lambda i: (0, 0)),
            pl.BlockSpec((BM, D), lambda i: (i, 0)),
            pl.BlockSpec((BM, D), lambda i: (i, 0)),
            pl.BlockSpec((D, F), lambda i: (0, 0)),
            pl.BlockSpec((D, F), lambda i: (0, 0)),
            pl.BlockSpec((F, D), lambda i: (0, 0)),
        ],
        out_specs=pl.BlockSpec((BM, D), lambda i: (i, 0)),
        out_shape=jax.ShapeDtypeStruct((T, D), jnp.float32),
    )(hg, wd_stack_bf16, h, x2b, ws_g, ws_u, ws_d)


def kernel(hidden_states, positions, input_ln_w, post_ln_w, W_qkv_a,
           q_a_ln_w, W_q_b, kv_a_ln_w, W_kv_b, W_o, W_router, e_bias,
           W_g, W_u, W_d, Ws_g, Ws_u, Ws_d):
    inv = 1.0 / (10000.0 ** (jnp.arange(0, DR, 2, dtype=jnp.float32) / DR))
    ang = positions.astype(jnp.float32)[:, None] * inv[None, :]
    cos2 = jnp.repeat(jnp.cos(ang), 2, axis=1)
    sin2 = jnp.repeat(jnp.sin(ang), 2, axis=1)

    qkv = _mm_rms(hidden_states, input_ln_w, W_qkv_a)
    q = _mm_rms(qkv, q_a_ln_w, W_q_b, kwidth=QLR, colblk=0)
    kv = _mm_rms(qkv, kv_a_ln_w, W_kv_b, kwidth=KVLR, colblk=QLR // KVLR)

    o_cat = _attention(q, qkv[:, QLR + KVLR:], kv, cos2, sin2)
    h = _wo_res(o_cat, W_o, hidden_states)

    x2b, sel, wfull = _router(h, post_ln_w, W_router, e_bias)
    keepw = _capacity(sel, wfull)

    hg = _experts_hg(x2b, W_g, W_u, keepw)
    wd_stack = W_d.reshape(E * F, D).astype(jnp.bfloat16)
    out = _final(hg, wd_stack, h, x2b, Ws_g, Ws_u, Ws_d)
    return out
